# baseline (speedup 1.0000x reference)
import functools

import jax
import jax.numpy as jnp
from jax import lax
from jax.experimental import pallas as pl
from jax.experimental.pallas import tpu as pltpu
from jax.experimental.pallas import tpu_sc as plsc

N_NODES = 10000
D = 256
N_GRAPHS = 512


NC, NS = 2, 16
NW = NC * NS
LANES = 16
CHUNK = 80
NCHUNKS = N_NODES // CHUNK
ITERS = -(-NCHUNKS // NW)

_sc_mesh = plsc.VectorSubcoreMesh(core_axis_name="c", subcore_axis_name="s")


PAIRS = ITERS // 2


@functools.partial(
    pl.kernel,
    out_type=jax.ShapeDtypeStruct((N_NODES, D), jnp.float32),
    mesh=_sc_mesh,
    scratch_types=[
        pltpu.VMEM((NCHUNKS, 1, CHUNK), jnp.int32),
        pltpu.VMEM((CHUNK, D), jnp.float32),
        pltpu.VMEM((CHUNK, D), jnp.float32),
        pltpu.VMEM((CHUNK, D), jnp.float32),
        pltpu.VMEM((CHUNK, D), jnp.float32),
        pltpu.SemaphoreType.DMA,
        pltpu.SemaphoreType.DMA,
    ],
)
def _sc_gather_add(x_hbm, batch_hbm, vx_hbm, h_hbm, idx_v,
                   rows_v0, rows_v1, x_v0, x_v1, sem_in0, sem_in1):
    cid = lax.axis_index("c")
    sid = lax.axis_index("s")
    wid = cid * NS + sid

    rows_v = [rows_v0, rows_v1]
    x_v = [x_v0, x_v1]
    sem_in = [sem_in0, sem_in1]

    pltpu.sync_copy(batch_hbm, idx_v)

    def issue_in(k, b):
        base = k * CHUNK
        pltpu.async_copy(x_hbm.at[pl.ds(base, CHUNK)], x_v[b], sem_in[b])
        pltpu.async_copy(vx_hbm.at[idx_v.at[k].at[0]], rows_v[b], sem_in[b])

    def drain_in(b):
        pltpu.make_async_copy(x_hbm.at[pl.ds(0, CHUNK)], x_v[b], sem_in[b]).wait()
        pltpu.make_async_copy(x_hbm.at[pl.ds(0, CHUNK)], rows_v[b], sem_in[b]).wait()

    def fin(k, b):
        @pl.loop(0, CHUNK, unroll=2)
        def _(i):
            for j in range(0, D, LANES):
                slc = (pl.ds(i, 1), pl.ds(j, LANES))
                rows_v[b].at[*slc][...] = (
                    rows_v[b].at[*slc][...] + x_v[b].at[*slc][...])

        pltpu.sync_copy(rows_v[b], h_hbm.at[pl.ds(k * CHUNK, CHUNK)])

    issue_in(wid, 0)

    @pl.loop(0, PAIRS)
    def _(t):
        kA = wid + (2 * t) * NW
        kB = kA + NW
        kN = kA + 2 * NW

        @pl.when(kB < NCHUNKS)
        def _():
            issue_in(kB, 1)

        drain_in(0)
        fin(kA, 0)

        @pl.when(kN < NCHUNKS)
        def _():
            issue_in(kN, 0)

        @pl.when(kB < NCHUNKS)
        def _():
            drain_in(1)
            fin(kB, 1)


BLOCK = 400
GRID = N_NODES // BLOCK


def _tc_body(x_ref, batch_ref, vx_ref, W0_ref, W1_ref, bsum_ref, s_ref, t_ref,
             v_ref, pool_acc, cnt_acc):
    i = pl.program_id(0)

    ids = batch_ref[0, 0, :]
    M = (ids[:, None] == lax.broadcasted_iota(jnp.int32, (BLOCK, N_GRAPHS), 1)
         ).astype(jnp.float32)

    part = lax.dot_general(M, x_ref[...], (((0,), (0,)), ((), ())),
                           preferred_element_type=jnp.float32)
    cnt = jnp.sum(M, axis=0).reshape(N_GRAPHS, 1)

    @pl.when(i == 0)
    def _():
        pool_acc[...] = part
        cnt_acc[...] = cnt

    @pl.when(i > 0)
    def _():
        pool_acc[...] += part
        cnt_acc[...] += cnt

    @pl.when(i == GRID - 1)
    def _():
        pooled = pool_acc[...] + cnt_acc[...] * vx_ref[...]
        A = lax.dot_general(vx_ref[...], W0_ref[...], (((1,), (1,)), ((), ())),
                            preferred_element_type=jnp.float32)
        P = lax.dot_general(pooled, W1_ref[...], (((1,), (1,)), ((), ())),
                            preferred_element_type=jnp.float32)
        v = (A + P + bsum_ref[...]) * s_ref[...] + t_ref[...]
        v_ref[...] = jnp.maximum(v, 0.0)


def kernel(x, edge_index, batch, vx, W0_w, W0_b, W1_w, W1_b,
           bn_gamma, bn_beta, bn_mean, bn_var):
    del edge_index
    s = bn_gamma * lax.rsqrt(bn_var + 1e-5)
    t = bn_beta - bn_mean * s
    bsum = (W0_b + W1_b).reshape(1, D)
    batch3 = batch.reshape(GRID, 1, BLOCK)

    v = pl.pallas_call(
        _tc_body,
        grid=(GRID,),
        in_specs=[
            pl.BlockSpec((BLOCK, D), lambda i: (i, 0)),
            pl.BlockSpec((1, 1, BLOCK), lambda i: (i, 0, 0)),
            pl.BlockSpec((N_GRAPHS, D), lambda i: (0, 0)),
            pl.BlockSpec((D, D), lambda i: (0, 0)),
            pl.BlockSpec((D, D), lambda i: (0, 0)),
            pl.BlockSpec((1, D), lambda i: (0, 0)),
            pl.BlockSpec((1, D), lambda i: (0, 0)),
            pl.BlockSpec((1, D), lambda i: (0, 0)),
        ],
        out_specs=pl.BlockSpec((N_GRAPHS, D), lambda i: (0, 0)),
        out_shape=jax.ShapeDtypeStruct((N_GRAPHS, D), jnp.float32),
        scratch_shapes=[
            pltpu.VMEM((N_GRAPHS, D), jnp.float32),
            pltpu.VMEM((N_GRAPHS, 1), jnp.float32),
        ],
    )(x, batch3, vx, W0_w, W1_w, bsum, s.reshape(1, D), t.reshape(1, D))
    h = _sc_gather_add(x, batch.reshape(NCHUNKS, 1, CHUNK), vx)
    return (h, v)
